# baseline (speedup 1.0000x reference)
import functools

import jax
import jax.numpy as jnp
from jax import lax
from jax.experimental import pallas as pl
from jax.experimental.pallas import tpu as pltpu
from jax.experimental.pallas import tpu_sc as plsc

BATCH = 16384
EMBED_DIM = 16
LANES = 16
NUM_CORES = 2
NUM_SUBCORES = 16
NW = NUM_CORES * NUM_SUBCORES
BPW = BATCH // NW
GROUPS = BPW // LANES
ROWS_PER_LINE = 128 // EMBED_DIM


def _sc_partial_sums(input_user, input_item, pred_data_label,
                     user_lines, item_lines, bias_tab):
  mesh = plsc.VectorSubcoreMesh(core_axis_name="c", subcore_axis_name="s")

  @functools.partial(
      pl.kernel,
      out_type=[
          jax.ShapeDtypeStruct((NW * LANES,), jnp.float32),
          jax.ShapeDtypeStruct((NW * LANES,), jnp.float32),
      ],
      mesh=mesh,
      compiler_params=pltpu.CompilerParams(needs_layout_passes=False),
      scratch_types=[
          pltpu.VMEM((BPW,), jnp.int32),
          pltpu.VMEM((BPW,), jnp.int32),
          pltpu.VMEM((BPW,), jnp.int32),
          pltpu.VMEM((BPW,), jnp.int32),
          pltpu.VMEM((BPW,), jnp.int32),
          pltpu.VMEM((BPW, 128), jnp.float32),
          pltpu.VMEM((EMBED_DIM, BPW), jnp.float32),
          pltpu.VMEM((BPW,), jnp.float32),
          pltpu.VMEM((BPW,), jnp.float32),
          pltpu.VMEM((LANES,), jnp.float32),
          pltpu.SemaphoreType.DMA,
          pltpu.SemaphoreType.DMA,
          pltpu.SemaphoreType.DMA,
      ],
  )
  def sc_kernel(uidx_hbm, iidx_hbm, gidx_hbm, ulines_hbm, ilines_hbm,
                btab_hbm, out_s1, out_s2,
                idx_u, idx_i, idx_g, lines_idx, sub_off, lines, u_vals,
                bias_i, bias_g, acc_st,
                sem_rows, sem_bi, sem_bg):
    wid = lax.axis_index("s") * NUM_CORES + lax.axis_index("c")
    base = wid * BPW

    pltpu.sync_copy(uidx_hbm.at[pl.ds(base, BPW)], idx_u)
    pltpu.sync_copy(iidx_hbm.at[pl.ds(base, BPW)], idx_i)
    pltpu.sync_copy(gidx_hbm.at[pl.ds(base, BPW)], idx_g)

    cbi = pltpu.async_copy(btab_hbm.at[idx_i], bias_i, sem_bi)
    cbg = pltpu.async_copy(btab_hbm.at[idx_g], bias_g, sem_bg)

    def split_indices(idx_ref):
      def body(t, _):
        iv = idx_ref[pl.ds(t * LANES, LANES)]
        lines_idx[pl.ds(t * LANES, LANES)] = lax.shift_right_logical(iv, 3)
        sub_off[pl.ds(t * LANES, LANES)] = lax.shift_left(
            jnp.bitwise_and(iv, 7), 4)
        return 0

      lax.fori_loop(0, GROUPS, body, 0)

    iota16 = lax.iota(jnp.int32, LANES)
    zero = jnp.zeros((LANES,), jnp.float32)

    split_indices(idx_u)
    pltpu.async_copy(ulines_hbm.at[lines_idx], lines, sem_rows).wait()

    def extract_u(g, _):
      jv = g * LANES + iota16
      sv = sub_off[pl.ds(g * LANES, LANES)]
      for c in range(EMBED_DIM):
        u_vals[c, pl.ds(g * LANES, LANES)] = plsc.load_gather(
            lines, [jv, sv + c])
      return 0

    lax.fori_loop(0, GROUPS, extract_u, 0)

    def dot_phase(idx_ref, tab_hbm):
      split_indices(idx_ref)
      pltpu.async_copy(tab_hbm.at[lines_idx], lines, sem_rows).wait()

      def body(g, acc):
        jv = g * LANES + iota16
        sv = sub_off[pl.ds(g * LANES, LANES)]
        for c in range(EMBED_DIM):
          v = plsc.load_gather(lines, [jv, sv + c])
          acc = acc + v * u_vals[c, pl.ds(g * LANES, LANES)]
        return acc

      return lax.fori_loop(0, GROUPS, body, zero)

    acc1 = dot_phase(idx_i, ilines_hbm)
    acc2 = dot_phase(idx_g, ilines_hbm)

    cbi.wait()
    cbg.wait()

    def bias_body(t, carry):
      b1, b2 = carry
      return (b1 + bias_i[pl.ds(t * LANES, LANES)],
              b2 + bias_g[pl.ds(t * LANES, LANES)])

    b1, b2 = lax.fori_loop(0, GROUPS, bias_body, (zero, zero))

    acc_st[...] = acc1 + b1
    pltpu.sync_copy(acc_st, out_s1.at[pl.ds(wid * LANES, LANES)])
    acc_st[...] = acc2 + b2
    pltpu.sync_copy(acc_st, out_s2.at[pl.ds(wid * LANES, LANES)])

  return sc_kernel(input_user, input_item, pred_data_label,
                   user_lines, item_lines, bias_tab)


def _tc_loss(s1_partials, s2_partials):
  def body(s1_ref, s2_ref, out_ref):
    inv_b = 1.0 / float(BATCH)
    s1 = jnp.sum(s1_ref[...]) * inv_b
    s2 = jnp.sum(s2_ref[...]) * inv_b

    def softplus(x):
      return jnp.maximum(x, 0.0) + jnp.log(1.0 + jnp.exp(-jnp.abs(x)))

    out_ref[...] = jnp.full((1, 1), softplus(-s2) + softplus(s1))

  out = pl.pallas_call(
      body,
      out_shape=jax.ShapeDtypeStruct((1, 1), jnp.float32),
  )(s1_partials, s2_partials)
  return out[0, 0]


def kernel(input_user, input_item, pred_data_label,
           D_user_embeddings, D_item_embeddings, D_item_bias):
  n_users = D_user_embeddings.shape[0]
  n_items = D_item_embeddings.shape[0]
  user_lines = D_user_embeddings.reshape(n_users // ROWS_PER_LINE, 128)
  item_lines = D_item_embeddings.reshape(n_items // ROWS_PER_LINE, 128)
  s1, s2 = _sc_partial_sums(input_user, input_item, pred_data_label,
                            user_lines, item_lines, D_item_bias)
  return _tc_loss(s1.reshape(4, 128), s2.reshape(4, 128))
